# baseline (speedup 1.0000x reference)
import jax
import jax.numpy as jnp
from jax.experimental import pallas as pl

NUM_EXPERTS = 16
TOP_K = 2
BLOCK_T = 256


def _router_block(x_ref, w_ref, b_ref, o_ref):
    x = x_ref[...]
    w = w_ref[...]
    logits = jnp.dot(x, w, preferred_element_type=jnp.float32) + b_ref[...]

    m = jnp.max(logits, axis=-1, keepdims=True)
    e = jnp.exp(logits - m)
    p = e / jnp.sum(e, axis=-1, keepdims=True)

    ii = jax.lax.broadcasted_iota(jnp.int32, logits.shape, 1)
    i1 = jnp.min(jnp.where(logits == m, ii, NUM_EXPERTS), axis=-1, keepdims=True)
    l2 = jnp.where(ii == i1, -jnp.inf, logits)
    m2 = jnp.max(l2, axis=-1, keepdims=True)
    i2 = jnp.min(jnp.where(l2 == m2, ii, NUM_EXPERTS), axis=-1, keepdims=True)
    mask = (ii == i1) | (ii == i2)
    o_ref[...] = jnp.where(mask, p, 0.0)


def kernel(token_inputs, W, b, num_experts):
    B, S, D = token_inputs.shape
    E = W.shape[1]
    x = token_inputs.reshape(B * S, D)
    b2 = b.reshape(1, E)
    grid = (B * S // BLOCK_T,)
    out = pl.pallas_call(
        _router_block,
        grid=grid,
        in_specs=[
            pl.BlockSpec((BLOCK_T, D), lambda i: (i, 0)),
            pl.BlockSpec((D, E), lambda i: (0, 0)),
            pl.BlockSpec((1, E), lambda i: (0, 0)),
        ],
        out_specs=pl.BlockSpec((BLOCK_T, E), lambda i: (i, 0)),
        out_shape=jax.ShapeDtypeStruct((B * S, E), jnp.float32),
    )(x, W, b2)
    return out.reshape(B, S, E)


# speedup vs baseline: 1.3795x; 1.3795x over previous
import jax
import jax.numpy as jnp
from jax.experimental import pallas as pl

NUM_EXPERTS = 16
TOP_K = 2
BLOCK_T = 1024


def _router_block(x_ref, w_ref, b_ref, o_ref):
    x = x_ref[...]
    w = w_ref[...]
    logits = jnp.dot(x, w, preferred_element_type=jnp.float32) + b_ref[...]

    m = jnp.max(logits, axis=-1, keepdims=True)
    e = jnp.exp(logits - m)
    p = e / jnp.sum(e, axis=-1, keepdims=True)

    ii = jax.lax.broadcasted_iota(jnp.int32, logits.shape, 1)
    i1 = jnp.min(jnp.where(logits == m, ii, NUM_EXPERTS), axis=-1, keepdims=True)
    l2 = jnp.where(ii == i1, -jnp.inf, logits)
    m2 = jnp.max(l2, axis=-1, keepdims=True)
    i2 = jnp.min(jnp.where(l2 == m2, ii, NUM_EXPERTS), axis=-1, keepdims=True)
    mask = (ii == i1) | (ii == i2)
    o_ref[...] = jnp.where(mask, p, 0.0)


def kernel(token_inputs, W, b, num_experts):
    B, S, D = token_inputs.shape
    E = W.shape[1]
    x = token_inputs.reshape(B * S, D)
    b2 = b.reshape(1, E)
    grid = (B * S // BLOCK_T,)
    out = pl.pallas_call(
        _router_block,
        grid=grid,
        in_specs=[
            pl.BlockSpec((BLOCK_T, D), lambda i: (i, 0)),
            pl.BlockSpec((D, E), lambda i: (0, 0)),
            pl.BlockSpec((1, E), lambda i: (0, 0)),
        ],
        out_specs=pl.BlockSpec((BLOCK_T, E), lambda i: (i, 0)),
        out_shape=jax.ShapeDtypeStruct((B * S, E), jnp.float32),
    )(x, W, b2)
    return out.reshape(B, S, E)


# speedup vs baseline: 1.4611x; 1.0592x over previous
import jax
import jax.numpy as jnp
from jax.experimental import pallas as pl

NUM_EXPERTS = 16
TOP_K = 2
BLOCK_T = 1024


def _router_block(x_ref, w_ref, b_ref, o_ref):
    x = x_ref[...]
    w = w_ref[...]
    logits = jnp.dot(x, w, preferred_element_type=jnp.float32) + b_ref[...]

    m = jnp.max(logits, axis=-1, keepdims=True)
    e = jnp.exp(logits - m)
    p = e * (1.0 / jnp.sum(e, axis=-1, keepdims=True))

    ii = jax.lax.broadcasted_iota(jnp.int32, logits.shape, 1)
    i1 = jnp.argmax(logits, axis=-1, keepdims=True)
    sel1 = ii == i1
    i2 = jnp.argmax(jnp.where(sel1, -jnp.inf, logits), axis=-1, keepdims=True)
    mask = sel1 | (ii == i2)
    o_ref[...] = jnp.where(mask, p, 0.0)


def kernel(token_inputs, W, b, num_experts):
    B, S, D = token_inputs.shape
    E = W.shape[1]
    x = token_inputs.reshape(B * S, D)
    b2 = b.reshape(1, E)
    grid = (B * S // BLOCK_T,)
    out = pl.pallas_call(
        _router_block,
        grid=grid,
        in_specs=[
            pl.BlockSpec((BLOCK_T, D), lambda i: (i, 0)),
            pl.BlockSpec((D, E), lambda i: (0, 0)),
            pl.BlockSpec((1, E), lambda i: (0, 0)),
        ],
        out_specs=pl.BlockSpec((BLOCK_T, E), lambda i: (i, 0)),
        out_shape=jax.ShapeDtypeStruct((B * S, E), jnp.float32),
    )(x, W, b2)
    return out.reshape(B, S, E)
